# Initial kernel scaffold; baseline (speedup 1.0000x reference)
import jax
import jax.numpy as jnp
from jax.experimental import pallas as pl


def kernel(node_type, num_inverted_predecessors, edge_index, batch, W_enc, b_enc, Wl0, Wr0, b0, Wl1, Wr1, b1, g0, be0, g1, be1):
    raise NotImplementedError("write your pallas kernel here")


# speedup vs baseline: 5.7962x; 5.7962x over previous
import functools

import jax
import jax.numpy as jnp
from jax import lax
from jax.experimental import pallas as pl
from jax.experimental.pallas import tpu as pltpu
from jax.experimental.pallas import tpu_sc as plsc

N = 10000
E = 320000
D = 128
G = 64

NC = 2
NS = 16
NW = NC * NS
CH = 128
NB = (E + NW * CH - 1) // (NW * CH)
E_PAD = NW * CH * NB
N_PAD = 10112
RPT = N_PAD // NS


def _edge_pass_body(tab, srcr, dstr, zrows, out, src_cur, dst_cur,
                    rows_v, acc, sem, sem_i):
    c = lax.axis_index("c")
    s = lax.axis_index("s")
    wid = s * NC + c
    pltpu.sync_copy(zrows, acc.at[pl.ds(s * RPT, RPT)])
    plsc.subcore_barrier()

    def step(j, carry):
        ca = pltpu.async_copy(srcr.at[wid, j], src_cur, sem_i)
        cb = pltpu.async_copy(dstr.at[wid, j], dst_cur, sem_i)
        ca.wait()
        cb.wait()
        pltpu.async_copy(tab.at[src_cur], rows_v, sem).wait()
        pltpu.sync_copy(rows_v, acc.at[dst_cur], add=True)
        return carry

    lax.fori_loop(0, NB, step, 0)
    plsc.subcore_barrier()
    pltpu.sync_copy(acc.at[pl.ds(s * RPT, RPT)], out.at[c, pl.ds(s * RPT, RPT)])


def _make_edge_pass(d):
    mesh = plsc.VectorSubcoreMesh(core_axis_name="c", subcore_axis_name="s")
    return pl.kernel(
        _edge_pass_body,
        out_type=jax.ShapeDtypeStruct((NC, N_PAD, d), jnp.float32),
        mesh=mesh,
        compiler_params=pltpu.CompilerParams(use_tc_tiling_on_sc=(d % 128 == 0)),
        scratch_types=[
            pltpu.VMEM((CH,), jnp.int32),
            pltpu.VMEM((CH,), jnp.int32),
            pltpu.VMEM((CH, d), jnp.float32),
            pltpu.VMEM_SHARED((N_PAD, d), jnp.float32),
            pltpu.SemaphoreType.DMA,
            pltpu.SemaphoreType.DMA,
        ],
    )


def _k2_body(s2, x8, wx8, wl0, wr0, b0, g0, be0, h1_out, inv_out):
    s = s2[0] + s2[1]
    cnt = s[:, 2:3]
    inv = 1.0 / jnp.maximum(cnt, 1.0)
    gl = s * inv
    ml = jnp.dot(wx8[...], wl0[...], preferred_element_type=jnp.float32)
    mr = jnp.dot(wx8[...], wr0[...], preferred_element_type=jnp.float32)
    pre = (jnp.dot(gl, ml, preferred_element_type=jnp.float32)
           + jnp.dot(x8[...], mr, preferred_element_type=jnp.float32) + b0[...])
    rows = lax.broadcasted_iota(jnp.int32, (N_PAD, 1), 0)
    maskf = (rows < N).astype(jnp.float32)
    mu = jnp.sum(pre * maskf, axis=0, keepdims=True) / N
    cen = (pre - mu) * maskf
    var = jnp.sum(cen * cen, axis=0, keepdims=True) / N
    h = (pre - mu) / jnp.sqrt(var + 1e-5) * g0[...] + be0[...]
    h1_out[...] = jnp.maximum(h, 0.0) * maskf
    inv_out[...] = inv


def _k4_body(m2, h1, inv, batchr, batchc, wl1, wr1, b1, g1, be1, out):
    agg = (m2[0] + m2[1]) * inv[...]
    pre = (jnp.dot(agg, wl1[...], preferred_element_type=jnp.float32)
           + jnp.dot(h1[...], wr1[...], preferred_element_type=jnp.float32)
           + b1[...])
    rows = lax.broadcasted_iota(jnp.int32, (N_PAD, 1), 0)
    maskf = (rows < N).astype(jnp.float32)
    mu = jnp.sum(pre * maskf, axis=0, keepdims=True) / N
    cen = (pre - mu) * maskf
    var = jnp.sum(cen * cen, axis=0, keepdims=True) / N
    h2 = (pre - mu) / jnp.sqrt(var + 1e-5) * g1[...] + be1[...]

    bid = batchr[...]
    giota = lax.broadcasted_iota(jnp.int32, (G, N_PAD), 0)
    oht = (giota == bid).astype(jnp.float32)
    sums = jnp.dot(oht, h2, preferred_element_type=jnp.float32)
    cnt_g = jnp.sum(oht, axis=1, keepdims=True)
    out[:, pl.ds(0, D)] = sums / jnp.maximum(cnt_g, 1.0)

    bcol = batchc[...]
    gcol = lax.broadcasted_iota(jnp.int32, (G, 1), 0)

    def gstep(g, acc):
        msk = bcol == g
        m = jnp.max(jnp.where(msk, h2, -jnp.inf), axis=0, keepdims=True)
        nonempty = jnp.sum(msk.astype(jnp.float32)) > 0.0
        m = jnp.where(nonempty, m, 0.0)
        return jnp.where(gcol == g, m, acc)

    mx = lax.fori_loop(0, G, gstep, jnp.zeros((G, D), jnp.float32))
    out[:, pl.ds(D, D)] = mx


_k2 = pl.pallas_call(
    _k2_body,
    out_shape=[jax.ShapeDtypeStruct((N_PAD, D), jnp.float32),
               jax.ShapeDtypeStruct((N_PAD, 1), jnp.float32)],
)

_k4 = pl.pallas_call(
    _k4_body,
    out_shape=jax.ShapeDtypeStruct((G, 2 * D), jnp.float32),
)


@jax.jit
def kernel(node_type, num_inverted_predecessors, edge_index, batch,
           W_enc, b_enc, Wl0, Wr0, b0, Wl1, Wr1, b1, g0, be0, g1, be1):
    f32 = jnp.float32
    nt = node_type.astype(f32)
    ip = num_inverted_predecessors.astype(f32)
    ones_n = jnp.ones((N,), f32)
    zeros_n = jnp.zeros((N,), f32)

    padrow = jnp.zeros((N_PAD - N, 8), f32)
    t8 = jnp.concatenate([jnp.stack([nt, ip, ones_n, zeros_n, zeros_n, zeros_n, zeros_n, zeros_n],
                                    axis=1), padrow], axis=0)
    x8 = t8
    src = jnp.concatenate([edge_index[0], jnp.full((E_PAD - E,), N, jnp.int32)])
    dst = jnp.concatenate([edge_index[1], jnp.full((E_PAD - E,), N, jnp.int32)])
    srcr = src.reshape(NW, NB, CH)
    dstr = dst.reshape(NW, NB, CH)
    wx8 = jnp.concatenate([W_enc, b_enc.reshape(1, D), jnp.zeros((5, D), f32)], axis=0)
    batchp = jnp.concatenate([batch.astype(jnp.int32), jnp.full((N_PAD - N,), G, jnp.int32)])
    batchr = batchp.reshape(1, N_PAD)
    batchc = batchp.reshape(N_PAD, 1)
    z8 = jnp.zeros((RPT, 8), f32)
    z128 = jnp.zeros((RPT, D), f32)

    s2 = _make_edge_pass(8)(t8, srcr, dstr, z8)
    h1, inv = _k2(s2, x8, wx8, Wl0, Wr0, b0.reshape(1, D), g0.reshape(1, D), be0.reshape(1, D))
    m2 = _make_edge_pass(D)(h1, srcr, dstr, z128)
    out = _k4(m2, h1, inv, batchr, batchc, Wl1, Wr1, b1.reshape(1, D), g1.reshape(1, D), be1.reshape(1, D))
    return out
